# Initial kernel scaffold; baseline (speedup 1.0000x reference)
import jax
import jax.numpy as jnp
from jax.experimental import pallas as pl


def kernel(x, neighbors, W0, b0, W1, b1):
    raise NotImplementedError("write your pallas kernel here")


# speedup vs baseline: 1.7306x; 1.7306x over previous
import functools

import jax
import jax.numpy as jnp
from jax import lax
from jax.experimental import pallas as pl
from jax.experimental.pallas import tpu as pltpu
from jax.experimental.pallas import tpu_sc as plsc

_N = 100000
_D = 128
_NW = 32
_B = 128
_CHUNKS = 25
_NPAD = _NW * _B * _CHUNKS


_BM = 1024


def _mm_body(x_ref, wc_ref, b_ref, o0, o1, o2, o3, o4):
    y = jnp.dot(x_ref[...], wc_ref[...], preferred_element_type=jnp.float32)
    o0[...] = y[:, 0 * _D:1 * _D] + b_ref[...]
    o1[...] = y[:, 1 * _D:2 * _D]
    o2[...] = y[:, 2 * _D:3 * _D]
    o3[...] = y[:, 3 * _D:4 * _D]
    o4[...] = y[:, 4 * _D:5 * _D]


def _tc_tables(xp, wc, b):
    grid = _NPAD // _BM
    out_sd = jax.ShapeDtypeStruct((_NPAD, _D), jnp.float32)
    obs = pl.BlockSpec((_BM, _D), lambda i: (i, 0))
    return pl.pallas_call(
        _mm_body,
        grid=(grid,),
        in_specs=[
            pl.BlockSpec((_BM, _D), lambda i: (i, 0)),
            pl.BlockSpec((_D, 5 * _D), lambda i: (0, 0)),
            pl.BlockSpec((1, _D), lambda i: (0, 0)),
        ],
        out_specs=[obs, obs, obs, obs, obs],
        out_shape=[out_sd, out_sd, out_sd, out_sd, out_sd],
    )(xp, wc, b)


def _sc_body(y0_hbm, y1_hbm, y2_hbm, y3_hbm, y4_hbm,
             i0_hbm, i1_hbm, i2_hbm, i3_hbm,
             out_hbm,
             i0_v, i1_v, i2_v, i3_v,
             acc_v, g0_v, g1_v, g2_v, g3_v,
             s0, s1, s2, s3):
    wid = lax.axis_index("s") * 2 + lax.axis_index("c")
    base0 = wid * (_CHUNKS * _B)

    def chunk_body(ci, carry):
        base = base0 + ci * _B
        pltpu.sync_copy(i0_hbm.at[pl.ds(base, _B)], i0_v)
        pltpu.sync_copy(i1_hbm.at[pl.ds(base, _B)], i1_v)
        pltpu.sync_copy(i2_hbm.at[pl.ds(base, _B)], i2_v)
        pltpu.sync_copy(i3_hbm.at[pl.ds(base, _B)], i3_v)
        d0 = pltpu.async_copy(y1_hbm.at[i0_v], g0_v, s0)
        d1 = pltpu.async_copy(y2_hbm.at[i1_v], g1_v, s1)
        d2 = pltpu.async_copy(y3_hbm.at[i2_v], g2_v, s2)
        d3 = pltpu.async_copy(y4_hbm.at[i3_v], g3_v, s3)
        pltpu.sync_copy(y0_hbm.at[pl.ds(base, _B)], acc_v)
        d0.wait()
        d1.wait()
        d2.wait()
        d3.wait()

        def row_body(r, rcarry):
            for c in range(_D // 16):
                s = pl.ds(c * 16, 16)
                v = (acc_v[r, s] + g0_v[r, s] + g1_v[r, s]
                     + g2_v[r, s] + g3_v[r, s])
                acc_v[r, s] = jnp.maximum(v, 0.0)
            return rcarry

        lax.fori_loop(0, _B, row_body, 0)
        pltpu.sync_copy(acc_v, out_hbm.at[pl.ds(base, _B)])
        return carry

    lax.fori_loop(0, _CHUNKS, chunk_body, 0)


@functools.cache
def _sc_gather_sum_kernel():
    return pl.kernel(
        _sc_body,
        mesh=plsc.VectorSubcoreMesh(core_axis_name="c", subcore_axis_name="s"),
        out_type=jax.ShapeDtypeStruct((_NPAD, _D), jnp.float32),
        scratch_types=[
            pltpu.VMEM((_B,), jnp.int32),
            pltpu.VMEM((_B,), jnp.int32),
            pltpu.VMEM((_B,), jnp.int32),
            pltpu.VMEM((_B,), jnp.int32),
            pltpu.VMEM((_B, _D), jnp.float32),
            pltpu.VMEM((_B, _D), jnp.float32),
            pltpu.VMEM((_B, _D), jnp.float32),
            pltpu.VMEM((_B, _D), jnp.float32),
            pltpu.VMEM((_B, _D), jnp.float32),
            pltpu.SemaphoreType.DMA,
            pltpu.SemaphoreType.DMA,
            pltpu.SemaphoreType.DMA,
            pltpu.SemaphoreType.DMA,
        ],
    )


def _sc_gather_sum(*args):
    return _sc_gather_sum_kernel()(*args)


def kernel(x, neighbors, W0, b0, W1, b1):
    xp = jnp.pad(x, ((0, _NPAD - _N), (0, 0)))
    nb = jnp.pad(neighbors.astype(jnp.int32), ((0, _NPAD - _N), (0, 0)))
    i0 = nb[:, 0]
    i1 = nb[:, 1]
    i2 = nb[:, 2]
    i3 = nb[:, 3]

    def wcat(W):
        return W.reshape(5, _D, _D).transpose(1, 0, 2).reshape(_D, 5 * _D)

    y = _tc_tables(xp, wcat(W0), b0.reshape(1, _D))
    h1 = _sc_gather_sum(y[0], y[1], y[2], y[3], y[4], i0, i1, i2, i3)
    y = _tc_tables(h1, wcat(W1), b1.reshape(1, _D))
    h2 = _sc_gather_sum(y[0], y[1], y[2], y[3], y[4], i0, i1, i2, i3)
    return h2[:_N]
